# Initial kernel scaffold; baseline (speedup 1.0000x reference)
import jax
import jax.numpy as jnp
from jax.experimental import pallas as pl


def kernel(feats, edge_index, edge_weight, W0, W1):
    raise NotImplementedError("write your pallas kernel here")


# speedup vs baseline: 8.0257x; 8.0257x over previous
import functools

import jax
import jax.numpy as jnp
from jax import lax
from jax.experimental import pallas as pl
from jax.experimental.pallas import tpu as pltpu
from jax.experimental.pallas import tpu_sc as plsc

N = 10000
NP = 10240
D = 128
NC, NS = 2, 16
NW = NC * NS
EB = 128
ZR = NP // NS


def _make_spmm(bpw):
    mesh = plsc.VectorSubcoreMesh(core_axis_name="c", subcore_axis_name="s")

    @functools.partial(
        pl.kernel,
        out_type=jax.ShapeDtypeStruct((NC, NP, D), jnp.float32),
        mesh=mesh,
        scratch_types=[
            pltpu.VMEM((bpw, EB), jnp.int32),
            pltpu.VMEM((bpw, EB), jnp.int32),
            pltpu.VMEM((bpw, EB), jnp.float32),
            pltpu.VMEM((EB, D), jnp.float32),
            pltpu.VMEM_SHARED((NP, D), jnp.float32),
            pltpu.SemaphoreType.DMA,
        ],
    )
    def spmm(x_hbm, src_hbm, dst_hbm, w_hbm, out_hbm,
             src_v, dst_v, w_v, rows_v, acc, sem):
        c = lax.axis_index("c")
        s = lax.axis_index("s")
        wid = c * NS + s
        blk0 = wid * bpw

        pltpu.sync_copy(src_hbm.at[pl.ds(blk0, bpw)], src_v)
        pltpu.sync_copy(dst_hbm.at[pl.ds(blk0, bpw)], dst_v)
        pltpu.sync_copy(w_hbm.at[pl.ds(blk0, bpw)], w_v)

        def zero_body(j, carry):
            for f in range(D // 16):
                rows_v[j, pl.ds(f * 16, 16)] = jnp.zeros((16,), jnp.float32)
            return carry
        lax.fori_loop(0, EB, zero_body, 0)
        zbase = s * ZR
        for k in range(ZR // EB):
            pltpu.sync_copy(rows_v, acc.at[pl.ds(zbase + k * EB, EB)])
        rem = ZR % EB
        if rem:
            pltpu.sync_copy(rows_v.at[pl.ds(0, rem)],
                            acc.at[pl.ds(zbase + (ZR // EB) * EB, rem)])
        plsc.subcore_barrier()

        def block_body(b, carry):
            pltpu.async_copy(x_hbm.at[src_v.at[b]], rows_v, sem).wait()

            def group_body(gi, carry2):
                wvec = w_v[b, pl.ds(gi * 16, 16)]
                for i in range(16):
                    wv = jnp.full((16,), wvec[i], jnp.float32)
                    for f in range(D // 16):
                        sl = pl.ds(f * 16, 16)
                        rows_v[gi * 16 + i, sl] = rows_v[gi * 16 + i, sl] * wv
                return carry2
            lax.fori_loop(0, EB // 16, group_body, 0)

            pltpu.sync_copy(rows_v, acc.at[dst_v.at[b]], add=True)
            return carry
        lax.fori_loop(0, bpw, block_body, 0)
        plsc.subcore_barrier()

        for k in range(ZR // EB):
            r0 = zbase + k * EB
            pltpu.sync_copy(acc.at[pl.ds(r0, EB)], rows_v)
            pltpu.sync_copy(rows_v, out_hbm.at[c, pl.ds(r0, EB)])
        if rem:
            r0 = zbase + (ZR // EB) * EB
            pltpu.sync_copy(acc.at[pl.ds(r0, rem)], rows_v.at[pl.ds(0, rem)])
            pltpu.sync_copy(rows_v.at[pl.ds(0, rem)],
                            out_hbm.at[c, pl.ds(r0, rem)])

    return spmm


def _fused_matmul(p, W0, W1):
    def body(p_ref, w0_ref, w1_ref, o_ref):
        a = p_ref[0] + p_ref[1]
        t = jnp.maximum(
            jnp.dot(a, w0_ref[...], preferred_element_type=jnp.float32), 0.0)
        o_ref[...] = jnp.dot(t, w1_ref[...],
                             preferred_element_type=jnp.float32)

    return pl.pallas_call(
        body,
        grid=(10,),
        in_specs=[
            pl.BlockSpec((2, NP // 10, D), lambda i: (0, i, 0)),
            pl.BlockSpec(W0.shape, lambda i: (0, 0)),
            pl.BlockSpec(W1.shape, lambda i: (0, 0)),
        ],
        out_specs=pl.BlockSpec((NP // 10, D), lambda i: (i, 0)),
        out_shape=jax.ShapeDtypeStruct((NP, D), jnp.float32),
    )(p, W0, W1)


def _relu_sum(p):
    def body(p_ref, o_ref):
        o_ref[...] = jnp.maximum(p_ref[0] + p_ref[1], 0.0)

    return pl.pallas_call(
        body,
        grid=(10,),
        in_specs=[pl.BlockSpec((2, NP // 10, D), lambda i: (0, i, 0))],
        out_specs=pl.BlockSpec((NP // 10, D), lambda i: (i, 0)),
        out_shape=jax.ShapeDtypeStruct((NP, D), jnp.float32),
    )(p)


def kernel(feats, edge_index, edge_weight, W0, W1):
    E = edge_weight.shape[0]
    src = edge_index[0].astype(jnp.int32)
    dst = edge_index[1].astype(jnp.int32)
    w = edge_weight.astype(jnp.float32)

    per_round = NW * EB
    bpw = (E + per_round - 1) // per_round
    bpw = (bpw + 7) // 8 * 8
    e_pad = bpw * per_round
    pad = e_pad - E
    if pad:
        fill = (jnp.arange(pad, dtype=jnp.int32) * 97) % N
        src = jnp.concatenate([src, fill])
        dst = jnp.concatenate([dst, fill])
        w = jnp.concatenate([w, jnp.zeros((pad,), jnp.float32)])
    srcs = src.reshape(e_pad // EB, EB)
    dsts = dst.reshape(e_pad // EB, EB)
    ws = w.reshape(e_pad // EB, EB)

    spmm = _make_spmm(bpw)
    a = spmm(feats, srcs, dsts, ws)
    g = _fused_matmul(a, W0, W1)
    b = spmm(g, srcs, dsts, ws)
    return _relu_sum(b)[:N]


# speedup vs baseline: 11.1838x; 1.3935x over previous
import functools

import jax
import jax.numpy as jnp
from jax import lax
from jax.experimental import pallas as pl
from jax.experimental.pallas import tpu as pltpu
from jax.experimental.pallas import tpu_sc as plsc

N = 10000
NP = 10112
D = 128
NC, NS = 2, 16
NW = NC * NS
EB = 80
R = 4
ZR = NP // NS


def _make_spmm(bpw):
    mesh = plsc.VectorSubcoreMesh(core_axis_name="c", subcore_axis_name="s")
    n_super = bpw // R

    @functools.partial(
        pl.kernel,
        out_type=jax.ShapeDtypeStruct((NC, NP, D), jnp.float32),
        mesh=mesh,
        scratch_types=[
            [pltpu.VMEM((EB,), jnp.int32) for _ in range(R)],
            [pltpu.VMEM((EB,), jnp.int32) for _ in range(R)],
            [pltpu.VMEM((EB,), jnp.float32) for _ in range(R)],
            [pltpu.VMEM((EB, D), jnp.float32) for _ in range(R)],
            pltpu.VMEM_SHARED((NP, D), jnp.float32),
            [pltpu.SemaphoreType.DMA for _ in range(2)],
            [pltpu.SemaphoreType.DMA for _ in range(2)],
            [pltpu.SemaphoreType.DMA for _ in range(2)],
        ],
    )
    def spmm(x_hbm, src_hbm, dst_hbm, w_hbm, out_hbm,
             src_v, dst_v, w_v, rows, acc, esem, gsem, ssem):
        c = lax.axis_index("c")
        s = lax.axis_index("s")
        wid = c * NS + s
        ebase = wid * bpw * EB

        def eoff(b):
            return pl.multiple_of(ebase + b * EB, EB)

        def issue_edge(b, slot):
            pltpu.async_copy(src_hbm.at[pl.ds(eoff(b), EB)], src_v[slot],
                             esem[slot % 2])
            pltpu.async_copy(dst_hbm.at[pl.ds(eoff(b), EB)], dst_v[slot],
                             esem[slot % 2])
            pltpu.async_copy(w_hbm.at[pl.ds(eoff(b), EB)], w_v[slot],
                             esem[slot % 2])

        def wait_edge(b, slot):
            pltpu.make_async_copy(src_hbm.at[pl.ds(eoff(b), EB)], src_v[slot],
                                  esem[slot % 2]).wait()
            pltpu.make_async_copy(dst_hbm.at[pl.ds(eoff(b), EB)], dst_v[slot],
                                  esem[slot % 2]).wait()
            pltpu.make_async_copy(w_hbm.at[pl.ds(eoff(b), EB)], w_v[slot],
                                  esem[slot % 2]).wait()

        def issue_gather(slot):
            pltpu.async_copy(x_hbm.at[src_v[slot]], rows[slot],
                             gsem[slot % 2])

        def wait_gather(slot):
            pltpu.make_async_copy(x_hbm.at[src_v[slot]], rows[slot],
                                  gsem[slot % 2]).wait()

        def issue_scatter(slot):
            pltpu.async_copy(rows[slot], acc.at[dst_v[slot]],
                             ssem[slot % 2], add=True)

        def wait_scatter(slot):
            pltpu.make_async_copy(rows[slot], acc.at[dst_v[slot]],
                                  ssem[slot % 2]).wait()

        issue_edge(0, 0)
        issue_edge(1, 1)
        wait_edge(0, 0)
        issue_gather(0)

        def zero_body(j, carry):
            for f in range(D // 16):
                rows[3][j, pl.ds(f * 16, 16)] = jnp.zeros((16,), jnp.float32)
            return carry
        lax.fori_loop(0, EB, zero_body, 0)
        zbase = s * ZR
        zchunks = [EB] * (ZR // EB) + ([ZR % EB] if ZR % EB else [])
        off = 0
        for sz in zchunks:
            pltpu.sync_copy(rows[3].at[pl.ds(0, sz)],
                            acc.at[pl.ds(pl.multiple_of(zbase + off, 8), sz)])
            off += sz
        plsc.subcore_barrier()

        def scale(b, slot):
            def group_body(gi, carry):
                wvec = w_v[slot][pl.ds(gi * 16, 16)]
                for i in range(16):
                    wv = jnp.full((16,), wvec[i], jnp.float32)
                    for f in range(D // 16):
                        sl = pl.ds(f * 16, 16)
                        rows[slot][gi * 16 + i, sl] = (
                            rows[slot][gi * 16 + i, sl] * wv)
                return carry
            lax.fori_loop(0, EB // 16, group_body, 0)

        def super_body(sb, carry):
            for o in range(R):
                b = sb * R + o
                if o >= 2:
                    wait_scatter(o - 2)
                else:
                    @pl.when(sb > 0)
                    def _():
                        wait_scatter((o - 2) % R)
                wait_gather(o)
                if o < 2:
                    issue_edge(b + 2, (o + 2) % R)
                else:
                    @pl.when(sb < n_super - 1)
                    def _():
                        issue_edge(b + 2, (o + 2) % R)
                if o < R - 1:
                    wait_edge(b + 1, o + 1)
                    issue_gather(o + 1)
                else:
                    @pl.when(sb < n_super - 1)
                    def _():
                        wait_edge(b + 1, 0)
                        issue_gather(0)
                scale(b, o)
                issue_scatter(o)
            return carry
        lax.fori_loop(0, n_super, super_body, 0)
        wait_scatter(R - 2)
        wait_scatter(R - 1)
        plsc.subcore_barrier()

        off = 0
        for k, sz in enumerate(zchunks):
            p = k % 2
            if k >= 2:
                pltpu.make_async_copy(
                    rows[p].at[pl.ds(0, zchunks[k - 2])],
                    out_hbm.at[c, pl.ds(pl.multiple_of(zbase, 8),
                                        zchunks[k - 2])],
                    gsem[p]).wait()
            r0 = pl.multiple_of(zbase + off, 8)
            pltpu.sync_copy(acc.at[pl.ds(r0, sz)], rows[p].at[pl.ds(0, sz)])
            pltpu.async_copy(rows[p].at[pl.ds(0, sz)],
                             out_hbm.at[c, pl.ds(r0, sz)], gsem[p])
            off += sz
        nchunk = len(zchunks)
        for k in (nchunk - 2, nchunk - 1):
            pltpu.make_async_copy(
                rows[k % 2].at[pl.ds(0, zchunks[k])],
                out_hbm.at[c, pl.ds(pl.multiple_of(zbase, 8), zchunks[k])],
                gsem[k % 2]).wait()

    return spmm


def _fused_matmul(p, W0, W1):
    def body(p_ref, w0_ref, w1_ref, o_ref):
        a = p_ref[0] + p_ref[1]
        t = jnp.maximum(
            jnp.dot(a, w0_ref[...], preferred_element_type=jnp.float32), 0.0)
        o_ref[...] = jnp.dot(t, w1_ref[...],
                             preferred_element_type=jnp.float32)

    return pl.pallas_call(
        body,
        grid=(8,),
        in_specs=[
            pl.BlockSpec((2, NP // 8, D), lambda i: (0, i, 0)),
            pl.BlockSpec(W0.shape, lambda i: (0, 0)),
            pl.BlockSpec(W1.shape, lambda i: (0, 0)),
        ],
        out_specs=pl.BlockSpec((NP // 8, D), lambda i: (i, 0)),
        out_shape=jax.ShapeDtypeStruct((NP, D), jnp.float32),
    )(p, W0, W1)


def _relu_sum(p):
    def body(p_ref, o_ref):
        o_ref[...] = jnp.maximum(p_ref[0] + p_ref[1], 0.0)

    return pl.pallas_call(
        body,
        grid=(8,),
        in_specs=[pl.BlockSpec((2, NP // 8, D), lambda i: (0, i, 0))],
        out_specs=pl.BlockSpec((NP // 8, D), lambda i: (i, 0)),
        out_shape=jax.ShapeDtypeStruct((NP, D), jnp.float32),
    )(p)


def kernel(feats, edge_index, edge_weight, W0, W1):
    E = edge_weight.shape[0]
    src = edge_index[0].astype(jnp.int32)
    dst = edge_index[1].astype(jnp.int32)
    w = edge_weight.astype(jnp.float32)

    per_round = NW * EB
    bpw = (E + per_round - 1) // per_round
    bpw = (bpw + R - 1) // R * R
    e_pad = bpw * per_round
    pad = e_pad - E
    if pad:
        fill = (jnp.arange(pad, dtype=jnp.int32) * 97) % N
        src = jnp.concatenate([src, fill])
        dst = jnp.concatenate([dst, fill])
        w = jnp.concatenate([w, jnp.zeros((pad,), jnp.float32)])

    spmm = _make_spmm(bpw)
    a = spmm(feats, src, dst, w)
    g = _fused_matmul(a, W0, W1)
    b = spmm(g, src, dst, w)
    return _relu_sum(b)[:N]


# speedup vs baseline: 13.9347x; 1.2460x over previous
import functools

import jax
import jax.numpy as jnp
from jax import lax
from jax.experimental import pallas as pl
from jax.experimental.pallas import tpu as pltpu
from jax.experimental.pallas import tpu_sc as plsc

N = 10000
D = 128
NC, NS = 2, 16
NW = NC * NS
EB = 48
RB = 6
GD = 3
ZR = 624


def _make_spmm(bpw):
    mesh = plsc.VectorSubcoreMesh(core_axis_name="c", subcore_axis_name="s")
    n_super = bpw // RB

    @functools.partial(
        pl.kernel,
        out_type=jax.ShapeDtypeStruct((NC, N, D), jnp.float32),
        mesh=mesh,
        scratch_types=[
            [pltpu.VMEM((EB,), jnp.int32) for _ in range(RB)],
            [pltpu.VMEM((EB,), jnp.int32) for _ in range(RB)],
            [pltpu.VMEM((EB,), jnp.float32) for _ in range(RB)],
            [pltpu.VMEM((EB, D), jnp.float32) for _ in range(RB)],
            pltpu.VMEM_SHARED((N, D), jnp.float32),
            [pltpu.SemaphoreType.DMA for _ in range(2)],
            [pltpu.SemaphoreType.DMA for _ in range(GD)],
            [pltpu.SemaphoreType.DMA for _ in range(2)],
        ],
    )
    def spmm(x_hbm, src_hbm, dst_hbm, w_hbm, out_hbm,
             src_v, dst_v, w_v, rows, acc, esem, gsem, ssem):
        c = lax.axis_index("c")
        s = lax.axis_index("s")
        wid = c * NS + s
        ebase = wid * bpw * EB

        def eoff(b):
            return pl.multiple_of(ebase + b * EB, EB)

        def issue_edge(b, slot):
            for ref, buf in ((src_hbm, src_v), (dst_hbm, dst_v), (w_hbm, w_v)):
                pltpu.async_copy(ref.at[pl.ds(eoff(b), EB)], buf[slot],
                                 esem[slot % 2])

        def wait_edge(b, slot):
            for ref, buf in ((src_hbm, src_v), (dst_hbm, dst_v), (w_hbm, w_v)):
                pltpu.make_async_copy(ref.at[pl.ds(eoff(b), EB)], buf[slot],
                                      esem[slot % 2]).wait()

        def issue_gather(slot):
            pltpu.async_copy(x_hbm.at[src_v[slot]], rows[slot],
                             gsem[slot % GD])

        def wait_gather(slot):
            pltpu.make_async_copy(x_hbm.at[src_v[slot]], rows[slot],
                                  gsem[slot % GD]).wait()

        def issue_scatter(slot):
            pltpu.async_copy(rows[slot], acc.at[dst_v[slot]],
                             ssem[slot % 2], add=True)

        def wait_scatter(slot):
            pltpu.make_async_copy(rows[slot], acc.at[dst_v[slot]],
                                  ssem[slot % 2]).wait()

        for x in range(GD):
            issue_edge(x, x)
            wait_edge(x, x)
            issue_gather(x)
        issue_edge(GD, GD)

        def zero_body(j, carry):
            for f in range(D // 16):
                rows[5][j, pl.ds(f * 16, 16)] = jnp.zeros((16,), jnp.float32)
            return carry
        lax.fori_loop(0, EB, zero_body, 0)
        zbase = s * ZR
        zchunks = [(k * EB, EB) for k in range(ZR // EB)]
        if ZR % EB:
            zchunks.append(((ZR // EB) * EB, ZR % EB))
        for off, sz in zchunks:
            pltpu.sync_copy(rows[5].at[pl.ds(0, sz)],
                            acc.at[pl.ds(pl.multiple_of(zbase + off, 8), sz)])

        @pl.when(s == NS - 1)
        def _():
            pltpu.sync_copy(rows[5].at[pl.ds(0, 16)],
                            acc.at[pl.ds(N - 16, 16)])
        plsc.subcore_barrier()

        def scale(slot):
            def group_body(gi, carry):
                wvec = w_v[slot][pl.ds(gi * 16, 16)]
                for i in range(16):
                    wv = jnp.full((16,), wvec[i], jnp.float32)
                    for f in range(D // 16):
                        sl = pl.ds(f * 16, 16)
                        rows[slot][gi * 16 + i, sl] = (
                            rows[slot][gi * 16 + i, sl] * wv)
                return carry
            lax.fori_loop(0, EB // 16, group_body, 0)

        def super_body(sb, carry):
            for o in range(RB):
                b = sb * RB + o
                if o >= 2:
                    wait_scatter(o - 2)
                else:
                    @pl.when(sb > 0)
                    def _():
                        wait_scatter((o - 2) % RB)
                if o < RB - GD:
                    wait_edge(b + GD, (o + GD) % RB)
                    issue_gather((o + GD) % RB)
                else:
                    @pl.when(sb < n_super - 1)
                    def _():
                        wait_edge(b + GD, (o + GD) % RB)
                        issue_gather((o + GD) % RB)
                wait_gather(o)
                if o < 2:
                    issue_edge(b + 4, (o + 4) % RB)
                else:
                    @pl.when(sb < n_super - 1)
                    def _():
                        issue_edge(b + 4, (o + 4) % RB)
                scale(o)
                issue_scatter(o)
            return carry
        lax.fori_loop(0, n_super, super_body, 0)
        wait_scatter(RB - 2)
        wait_scatter(RB - 1)
        plsc.subcore_barrier()

        wchunks = list(zchunks)

        for k, (off, sz) in enumerate(wchunks):
            p = k % 2
            if k >= 2:
                posz = wchunks[k - 2][1]
                pltpu.make_async_copy(
                    rows[p].at[pl.ds(0, posz)],
                    out_hbm.at[c, pl.ds(pl.multiple_of(zbase, 8), posz)],
                    gsem[p]).wait()
            r0 = pl.multiple_of(zbase + off, 8)
            pltpu.sync_copy(acc.at[pl.ds(r0, sz)], rows[p].at[pl.ds(0, sz)])
            pltpu.async_copy(rows[p].at[pl.ds(0, sz)],
                             out_hbm.at[c, pl.ds(r0, sz)], gsem[p])
        nch = len(wchunks)
        for k in (nch - 2, nch - 1):
            pltpu.make_async_copy(
                rows[k % 2].at[pl.ds(0, wchunks[k][1])],
                out_hbm.at[c, pl.ds(pl.multiple_of(zbase, 8), wchunks[k][1])],
                gsem[k % 2]).wait()

        @pl.when(s == NS - 1)
        def _():
            pltpu.sync_copy(acc.at[pl.ds(N - 16, 16)],
                            rows[2].at[pl.ds(0, 16)])
            pltpu.sync_copy(rows[2].at[pl.ds(0, 16)],
                            out_hbm.at[c, pl.ds(N - 16, 16)])

    return spmm


def _fused_matmul(p, W0, W1):
    def body(p_ref, w0_ref, w1_ref, o_ref):
        a = p_ref[0] + p_ref[1]
        t = jnp.maximum(
            jnp.dot(a, w0_ref[...], preferred_element_type=jnp.float32), 0.0)
        o_ref[...] = jnp.dot(t, w1_ref[...],
                             preferred_element_type=jnp.float32)

    return pl.pallas_call(
        body,
        grid=(10,),
        in_specs=[
            pl.BlockSpec((2, N // 10, D), lambda i: (0, i, 0)),
            pl.BlockSpec(W0.shape, lambda i: (0, 0)),
            pl.BlockSpec(W1.shape, lambda i: (0, 0)),
        ],
        out_specs=pl.BlockSpec((N // 10, D), lambda i: (i, 0)),
        out_shape=jax.ShapeDtypeStruct((N, D), jnp.float32),
    )(p, W0, W1)


def _relu_sum(p):
    def body(p_ref, o_ref):
        o_ref[...] = jnp.maximum(p_ref[0] + p_ref[1], 0.0)

    return pl.pallas_call(
        body,
        grid=(10,),
        in_specs=[pl.BlockSpec((2, N // 10, D), lambda i: (0, i, 0))],
        out_specs=pl.BlockSpec((N // 10, D), lambda i: (i, 0)),
        out_shape=jax.ShapeDtypeStruct((N, D), jnp.float32),
    )(p)


def kernel(feats, edge_index, edge_weight, W0, W1):
    E = edge_weight.shape[0]
    src = edge_index[0].astype(jnp.int32)
    dst = edge_index[1].astype(jnp.int32)
    w = edge_weight.astype(jnp.float32)

    per_round = NW * EB
    bpw = (E + per_round - 1) // per_round
    bpw = (bpw + RB - 1) // RB * RB
    e_pad = bpw * per_round
    pad = e_pad - E
    if pad:
        fill = (jnp.arange(pad, dtype=jnp.int32) * 97) % N
        src = jnp.concatenate([src, fill])
        dst = jnp.concatenate([dst, fill])
        w = jnp.concatenate([w, jnp.zeros((pad,), jnp.float32)])

    spmm = _make_spmm(bpw)
    a = spmm(feats, src, dst, w)
    g = _fused_matmul(a, W0, W1)
    b = spmm(g, src, dst, w)
    return _relu_sum(b)
